# Initial kernel scaffold; baseline (speedup 1.0000x reference)
import jax
import jax.numpy as jnp
from jax.experimental import pallas as pl


def kernel(z_e, embedding_weight):
    raise NotImplementedError("write your pallas kernel here")


# speedup vs baseline: 4.3296x; 4.3296x over previous
import functools

import jax
import jax.numpy as jnp
from jax import lax
from jax.experimental import pallas as pl
from jax.experimental.pallas import tpu as pltpu
from jax.experimental.pallas import tpu_sc as plsc

F = 256
K = 8192
N = 4096

BN = 256
BK = 2048
IT = N // BN
JT = K // BK

NC = 2
NS = 16
NW = NC * NS
BPW = N // NW


def _argmin_body(z_ref, e_ref, idx_ref, bestv_ref, besti_ref):
    j = pl.program_id(1)
    z = z_ref[...]
    e = e_ref[...]
    a = jnp.sum(z * z, axis=1, keepdims=True)
    c = jnp.sum(e * e, axis=0, keepdims=True)
    b = jnp.dot(z, e, preferred_element_type=jnp.float32)
    d = (a - 2.0 * b) + c
    m = jnp.min(d, axis=1, keepdims=True)
    cols = lax.broadcasted_iota(jnp.int32, d.shape, 1)
    loc = jnp.min(jnp.where(d == m, cols, K), axis=1, keepdims=True) + j * BK

    @pl.when(j == 0)
    def _():
        bestv_ref[...] = m
        besti_ref[...] = loc

    @pl.when(j > 0)
    def _():
        better = m < bestv_ref[...]
        bestv_ref[...] = jnp.where(better, m, bestv_ref[...])
        besti_ref[...] = jnp.where(better, loc, besti_ref[...])

    idx_ref[...] = besti_ref[...]


def _onehot_body(idx_ref, enc_ref):
    j = pl.program_id(1)
    cols = lax.broadcasted_iota(jnp.int32, (BN, BK), 1) + j * BK
    enc_ref[...] = (cols == idx_ref[...]).astype(jnp.float32)


def _sc_body(et_ref, idx_ref, ze_ref, zq_ref, diff_ref, idx_v, rows_v, ze_v, sem):
    wid = lax.axis_index("s") * NC + lax.axis_index("c")
    base = wid * BPW
    pltpu.sync_copy(idx_ref.at[pl.ds(base, BPW)], idx_v)
    cp = pltpu.async_copy(et_ref.at[idx_v], rows_v, sem)
    pltpu.sync_copy(ze_ref.at[pl.ds(base, BPW)], ze_v)
    cp.wait()

    def row(r, carry):
        for c0 in range(F // 16):
            sl = pl.ds(c0 * 16, 16)
            zq = rows_v[r, sl]
            ze = ze_v[r, sl]
            t = zq - ze
            rows_v[r, sl] = ze + t
            ze_v[r, sl] = t * t
        return carry

    lax.fori_loop(0, BPW, row, 0)
    pltpu.sync_copy(rows_v, zq_ref.at[pl.ds(base, BPW)])
    pltpu.sync_copy(ze_v, diff_ref.at[pl.ds(base, BPW)])


def _sc_quantise(e_t, idx_flat, z_e):
    run = functools.partial(
        pl.kernel,
        out_type=(jax.ShapeDtypeStruct((N, F), jnp.float32),
                  jax.ShapeDtypeStruct((N, F), jnp.float32)),
        mesh=plsc.VectorSubcoreMesh(core_axis_name="c", subcore_axis_name="s"),
        scratch_types=[
            pltpu.VMEM((BPW,), jnp.int32),
            pltpu.VMEM((BPW, F), jnp.float32),
            pltpu.VMEM((BPW, F), jnp.float32),
            pltpu.SemaphoreType.DMA,
        ],
    )(_sc_body)
    return run(e_t, idx_flat, z_e)


def kernel(z_e, embedding_weight):
    idx = pl.pallas_call(
        _argmin_body,
        grid=(IT, JT),
        in_specs=[
            pl.BlockSpec((BN, F), lambda i, j: (i, 0)),
            pl.BlockSpec((F, BK), lambda i, j: (0, j)),
        ],
        out_specs=pl.BlockSpec((BN, 1), lambda i, j: (i, 0)),
        out_shape=jax.ShapeDtypeStruct((N, 1), jnp.int32),
        scratch_shapes=[
            pltpu.VMEM((BN, 1), jnp.float32),
            pltpu.VMEM((BN, 1), jnp.int32),
        ],
    )(z_e, embedding_weight)

    enc = pl.pallas_call(
        _onehot_body,
        grid=(IT, JT),
        in_specs=[pl.BlockSpec((BN, 1), lambda i, j: (i, 0))],
        out_specs=pl.BlockSpec((BN, BK), lambda i, j: (i, j)),
        out_shape=jax.ShapeDtypeStruct((N, K), jnp.float32),
    )(idx)

    e_t = embedding_weight.T
    zq_st, diff = _sc_quantise(e_t, idx[:, 0], z_e)
    return (zq_st, idx, enc, diff)


# speedup vs baseline: 5.9379x; 1.3715x over previous
import functools

import jax
import jax.numpy as jnp
from jax import lax
from jax.experimental import pallas as pl
from jax.experimental.pallas import tpu as pltpu
from jax.experimental.pallas import tpu_sc as plsc

F = 256
K = 8192
N = 4096

BN = 256
BK = 2048
IT = N // BN
JT = K // BK

NC = 2
NS = 16
NW = NC * NS
BPW = N // NW


def _argmin_body(z_ref, e_ref, idx_ref):
    z = z_ref[...]
    e = e_ref[...]
    a = jnp.sum(z * z, axis=1, keepdims=True)
    c = jnp.sum(e * e, axis=0, keepdims=True)
    b = jnp.dot(z, e, preferred_element_type=jnp.float32)
    d = (a - 2.0 * b) + c
    m = jnp.min(d, axis=1, keepdims=True)
    cols = lax.broadcasted_iota(jnp.int32, d.shape, 1)
    idx_ref[...] = jnp.min(jnp.where(d == m, cols, K), axis=1, keepdims=True)


def _onehot_body(idx_ref, enc_ref):
    cols = lax.broadcasted_iota(jnp.int32, (BN, K), 1)
    enc_ref[...] = (cols == idx_ref[...]).astype(jnp.float32)


def _sc_body(et_ref, idx_ref, ze_ref, zq_ref, diff_ref, idx_v, rows_v, ze_v, sem):
    wid = lax.axis_index("s") * NC + lax.axis_index("c")
    base = wid * BPW
    pltpu.sync_copy(idx_ref.at[pl.ds(base, BPW)], idx_v)
    cp = pltpu.async_copy(et_ref.at[idx_v], rows_v, sem)
    pltpu.sync_copy(ze_ref.at[pl.ds(base, BPW)], ze_v)
    cp.wait()

    def row(r, carry):
        for c0 in range(F // 16):
            sl = pl.ds(c0 * 16, 16)
            zq = rows_v[r, sl]
            ze = ze_v[r, sl]
            t = zq - ze
            rows_v[r, sl] = ze + t
            ze_v[r, sl] = t * t
        return carry

    lax.fori_loop(0, BPW, row, 0)
    pltpu.sync_copy(rows_v, zq_ref.at[pl.ds(base, BPW)])
    pltpu.sync_copy(ze_v, diff_ref.at[pl.ds(base, BPW)])


def _sc_quantise(e_t, idx_flat, z_e):
    run = functools.partial(
        pl.kernel,
        out_type=(jax.ShapeDtypeStruct((N, F), jnp.float32),
                  jax.ShapeDtypeStruct((N, F), jnp.float32)),
        mesh=plsc.VectorSubcoreMesh(core_axis_name="c", subcore_axis_name="s"),
        scratch_types=[
            pltpu.VMEM((BPW,), jnp.int32),
            pltpu.VMEM((BPW, F), jnp.float32),
            pltpu.VMEM((BPW, F), jnp.float32),
            pltpu.SemaphoreType.DMA,
        ],
    )(_sc_body)
    return run(e_t, idx_flat, z_e)


def kernel(z_e, embedding_weight):
    idx = pl.pallas_call(
        _argmin_body,
        grid=(IT,),
        in_specs=[
            pl.BlockSpec((BN, F), lambda i: (i, 0)),
            pl.BlockSpec((F, K), lambda i: (0, 0)),
        ],
        out_specs=pl.BlockSpec((BN, 1), lambda i: (i, 0)),
        out_shape=jax.ShapeDtypeStruct((N, 1), jnp.int32),
    )(z_e, embedding_weight)

    enc = pl.pallas_call(
        _onehot_body,
        grid=(IT,),
        in_specs=[pl.BlockSpec((BN, 1), lambda i: (i, 0))],
        out_specs=pl.BlockSpec((BN, K), lambda i: (i, 0)),
        out_shape=jax.ShapeDtypeStruct((N, K), jnp.float32),
    )(idx)

    e_t = embedding_weight.T
    zq_st, diff = _sc_quantise(e_t, idx[:, 0], z_e)
    return (zq_st, idx, enc, diff)


# speedup vs baseline: 8.1384x; 1.3706x over previous
import functools

import jax
import jax.numpy as jnp
from jax import lax
from jax.experimental import pallas as pl
from jax.experimental.pallas import tpu as pltpu
from jax.experimental.pallas import tpu_sc as plsc

F = 256
K = 8192
N = 4096

BN = 256
BK = 2048
IT = N // BN
JT = K // BK

NC = 2
NS = 16
NW = NC * NS
BPW = N // NW


def _argmin_onehot_body(z_ref, e_ref, idx_ref, enc_ref, c_ref):
    i = pl.program_id(0)

    @pl.when(i == 0)
    def _():
        e = e_ref[...]
        c_ref[...] = jnp.sum(e * e, axis=0, keepdims=True)

    z = z_ref[...]
    a = jnp.sum(z * z, axis=1, keepdims=True)
    b2 = jnp.dot(z * (-2.0), e_ref[...], preferred_element_type=jnp.float32)
    d = (a + b2) + c_ref[...]
    m = jnp.min(d, axis=1, keepdims=True)
    cols = lax.broadcasted_iota(jnp.int32, d.shape, 1)
    loc = jnp.min(jnp.where(d == m, cols, K), axis=1, keepdims=True)
    idx_ref[...] = loc
    enc_ref[...] = (cols == loc).astype(jnp.float32)


def _sc_body(et_ref, idx_ref, ze_ref, zq_ref, diff_ref, idx_v, rows_v, ze_v, sem):
    wid = lax.axis_index("s") * NC + lax.axis_index("c")
    base = wid * BPW
    pltpu.sync_copy(idx_ref.at[pl.ds(base, BPW)], idx_v)
    cp = pltpu.async_copy(et_ref.at[idx_v], rows_v, sem)
    pltpu.sync_copy(ze_ref.at[pl.ds(base, BPW)], ze_v)
    cp.wait()

    def row(r, carry):
        for c0 in range(F // 16):
            sl = pl.ds(c0 * 16, 16)
            zq = rows_v[r, sl]
            ze = ze_v[r, sl]
            t = zq - ze
            rows_v[r, sl] = ze + t
            ze_v[r, sl] = t * t
        return carry

    lax.fori_loop(0, BPW, row, 0)
    pltpu.sync_copy(rows_v, zq_ref.at[pl.ds(base, BPW)])
    pltpu.sync_copy(ze_v, diff_ref.at[pl.ds(base, BPW)])


def _sc_quantise(e_t, idx_flat, z_e):
    run = functools.partial(
        pl.kernel,
        out_type=(jax.ShapeDtypeStruct((N, F), jnp.float32),
                  jax.ShapeDtypeStruct((N, F), jnp.float32)),
        mesh=plsc.VectorSubcoreMesh(core_axis_name="c", subcore_axis_name="s"),
        scratch_types=[
            pltpu.VMEM((BPW,), jnp.int32),
            pltpu.VMEM((BPW, F), jnp.float32),
            pltpu.VMEM((BPW, F), jnp.float32),
            pltpu.SemaphoreType.DMA,
        ],
    )(_sc_body)
    return run(e_t, idx_flat, z_e)


def kernel(z_e, embedding_weight):
    idx, enc = pl.pallas_call(
        _argmin_onehot_body,
        grid=(IT,),
        in_specs=[
            pl.BlockSpec((BN, F), lambda i: (i, 0)),
            pl.BlockSpec((F, K), lambda i: (0, 0)),
        ],
        out_specs=[
            pl.BlockSpec((BN, 1), lambda i: (i, 0)),
            pl.BlockSpec((BN, K), lambda i: (i, 0)),
        ],
        out_shape=[
            jax.ShapeDtypeStruct((N, 1), jnp.int32),
            jax.ShapeDtypeStruct((N, K), jnp.float32),
        ],
        scratch_shapes=[pltpu.VMEM((1, K), jnp.float32)],
    )(z_e, embedding_weight)

    e_t = embedding_weight.T
    zq_st, diff = _sc_quantise(e_t, idx[:, 0], z_e)
    return (zq_st, idx, enc, diff)


# speedup vs baseline: 8.4062x; 1.0329x over previous
import functools

import jax
import jax.numpy as jnp
from jax import lax
from jax.experimental import pallas as pl
from jax.experimental.pallas import tpu as pltpu
from jax.experimental.pallas import tpu_sc as plsc

F = 256
K = 8192
N = 4096

BN = 256
BK = 2048
IT = N // BN
JT = K // BK

NC = 2
NS = 16
NW = NC * NS
BPW = N // NW


def _argmin_onehot_body(z_ref, e_ref, idx_ref, enc_ref, c_ref, cols_ref):
    i = pl.program_id(0)

    @pl.when(i == 0)
    def _():
        e = e_ref[...]
        c_ref[...] = jnp.sum(e * e, axis=0, keepdims=True)
        cols_ref[...] = lax.broadcasted_iota(
            jnp.int32, (1, K), 1).astype(jnp.float32)

    z = z_ref[...]
    a = jnp.sum(z * z, axis=1, keepdims=True)
    b2 = jnp.dot(z * (-2.0), e_ref[...], preferred_element_type=jnp.float32)
    d = (a + b2) + c_ref[...]
    m = jnp.min(d, axis=1, keepdims=True)
    cols = cols_ref[...]
    loc = jnp.min(jnp.where(d == m, cols, float(K)), axis=1, keepdims=True)
    idx_ref[...] = loc.astype(jnp.int32)
    enc_ref[...] = (cols == loc).astype(jnp.float32)


def _sc_body(et_ref, idx_ref, ze_ref, zq_ref, diff_ref, idx_v, rows_v, ze_v, sem):
    wid = lax.axis_index("s") * NC + lax.axis_index("c")
    base = wid * BPW
    pltpu.sync_copy(idx_ref.at[pl.ds(base, BPW)], idx_v)
    cp = pltpu.async_copy(et_ref.at[idx_v], rows_v, sem)
    pltpu.sync_copy(ze_ref.at[pl.ds(base, BPW)], ze_v)
    cp.wait()

    def row(r, carry):
        for c0 in range(F // 16):
            sl = pl.ds(c0 * 16, 16)
            zq = rows_v[r, sl]
            ze = ze_v[r, sl]
            t = zq - ze
            rows_v[r, sl] = ze + t
            ze_v[r, sl] = t * t
        return carry

    lax.fori_loop(0, BPW, row, 0)
    pltpu.sync_copy(rows_v, zq_ref.at[pl.ds(base, BPW)])
    pltpu.sync_copy(ze_v, diff_ref.at[pl.ds(base, BPW)])


def _sc_quantise(e_t, idx_flat, z_e):
    run = functools.partial(
        pl.kernel,
        out_type=(jax.ShapeDtypeStruct((N, F), jnp.float32),
                  jax.ShapeDtypeStruct((N, F), jnp.float32)),
        mesh=plsc.VectorSubcoreMesh(core_axis_name="c", subcore_axis_name="s"),
        scratch_types=[
            pltpu.VMEM((BPW,), jnp.int32),
            pltpu.VMEM((BPW, F), jnp.float32),
            pltpu.VMEM((BPW, F), jnp.float32),
            pltpu.SemaphoreType.DMA,
        ],
    )(_sc_body)
    return run(e_t, idx_flat, z_e)


def kernel(z_e, embedding_weight):
    idx, enc = pl.pallas_call(
        _argmin_onehot_body,
        grid=(IT,),
        in_specs=[
            pl.BlockSpec((BN, F), lambda i: (i, 0)),
            pl.BlockSpec((F, K), lambda i: (0, 0)),
        ],
        out_specs=[
            pl.BlockSpec((BN, 1), lambda i: (i, 0)),
            pl.BlockSpec((BN, K), lambda i: (i, 0)),
        ],
        out_shape=[
            jax.ShapeDtypeStruct((N, 1), jnp.int32),
            jax.ShapeDtypeStruct((N, K), jnp.float32),
        ],
        scratch_shapes=[
            pltpu.VMEM((1, K), jnp.float32),
            pltpu.VMEM((1, K), jnp.float32),
        ],
    )(z_e, embedding_weight)

    e_t = embedding_weight.T
    zq_st, diff = _sc_quantise(e_t, idx[:, 0], z_e)
    return (zq_st, idx, enc, diff)
